# baseline (speedup 1.0000x reference)
import functools

import jax
import jax.numpy as jnp
from jax import lax
from jax.experimental import pallas as pl
from jax.experimental.pallas import tpu as pltpu
from jax.experimental.pallas import tpu_sc as plsc

N = 10000
E = 160000
F1 = 16
F2 = 8
NCLS = 10
NC = 2
NS = 16
NW = NC * NS
CHUNK = 128
EPAD = 163840
BPW = EPAD // (NW * CHUNK)
NPAD = 10240
ROWS_PER_SUB = NPAD // NS

_mesh = plsc.VectorSubcoreMesh(core_axis_name="c", subcore_axis_name="s")

HIGHEST = jax.lax.Precision.HIGHEST


_NRING = 8


@functools.partial(
    pl.kernel,
    out_type=jax.ShapeDtypeStruct((NC, NPAD), jnp.float32),
    mesh=_mesh,
    scratch_types=(
        [pltpu.VMEM((BPW * CHUNK,), jnp.float32)]
        + [pltpu.VMEM((CHUNK,), jnp.int32) for _ in range(_NRING)]
        + [pltpu.SemaphoreType.DMA for _ in range(2 * _NRING)]
        + [pltpu.VMEM_SHARED((NPAD,), jnp.float32)]
    ),
)
def _deg_kernel(dst_hbm, ew_hbm, zeros_hbm, out_hbm, ewbig, *rest):
    dbufs = rest[:_NRING]
    sds = rest[_NRING:2 * _NRING]
    sss = rest[2 * _NRING:3 * _NRING]
    acc_sh = rest[3 * _NRING]
    c = lax.axis_index("c")
    s = lax.axis_index("s")
    wid = c * NS + s
    ebase = wid * (BPW * CHUNK)

    pltpu.sync_copy(ew_hbm.at[pl.ds(ebase, BPW * CHUNK)], ewbig)
    pltpu.sync_copy(zeros_hbm.at[pl.ds(s * ROWS_PER_SUB, ROWS_PER_SUB)],
                    acc_sh.at[pl.ds(s * ROWS_PER_SUB, ROWS_PER_SUB)])
    plsc.subcore_barrier()

    def stage(i):
        b = i % _NRING
        pltpu.async_copy(dst_hbm.at[pl.ds(ebase + i * CHUNK, CHUNK)],
                         dbufs[b], sds[b])

    def wait_stage(i):
        b = i % _NRING
        pltpu.make_async_copy(dst_hbm.at[pl.ds(ebase + i * CHUNK, CHUNK)],
                              dbufs[b], sds[b]).wait()

    def scatter(i):
        b = i % _NRING
        pltpu.async_copy(ewbig.at[pl.ds(i * CHUNK, CHUNK)],
                         acc_sh.at[dbufs[b]], sss[b], add=True)

    def wait_scatter(i):
        b = i % _NRING
        pltpu.make_async_copy(ewbig.at[pl.ds(i * CHUNK, CHUNK)],
                              acc_sh.at[dbufs[b]], sss[b]).wait()

    for i in range(4):
        stage(i)
    for i in range(BPW):
        if i + 4 < BPW:
            if i - 4 >= 0:
                wait_scatter(i - 4)
            stage(i + 4)
        wait_stage(i)
        scatter(i)
    for i in range(BPW - _NRING, BPW):
        wait_scatter(i)

    plsc.subcore_barrier()
    pltpu.sync_copy(acc_sh.at[pl.ds(s * ROWS_PER_SUB, ROWS_PER_SUB)],
                    out_hbm.at[c, pl.ds(s * ROWS_PER_SUB, ROWS_PER_SUB)])


_ARING = 8


@functools.partial(
    pl.kernel,
    out_type=jax.ShapeDtypeStruct((NC, NPAD, F1), jnp.float32),
    mesh=_mesh,
    scratch_types=(
        [pltpu.VMEM((BPW * CHUNK,), jnp.int32),
         pltpu.VMEM((BPW * CHUNK,), jnp.float32)]
        + [pltpu.VMEM((CHUNK,), jnp.int32) for _ in range(_ARING)]
        + [pltpu.VMEM((CHUNK, F1), jnp.float32) for _ in range(_ARING)]
        + [pltpu.SemaphoreType.DMA for _ in range(3 * _ARING)]
        + [pltpu.VMEM_SHARED((NPAD, F1), jnp.float32)]
    ),
    compiler_params=pltpu.CompilerParams(use_tc_tiling_on_sc=False),
)
def _agg_kernel(g_hbm, src_hbm, dst_hbm, ew_hbm, zeros_hbm, out_hbm,
                srcbig, ewbig, *rest):
    dbufs = rest[:_ARING]
    rbufs = rest[_ARING:2 * _ARING]
    sds = rest[2 * _ARING:3 * _ARING]
    sgs = rest[3 * _ARING:4 * _ARING]
    sss = rest[4 * _ARING:5 * _ARING]
    acc_sh = rest[5 * _ARING]
    c = lax.axis_index("c")
    s = lax.axis_index("s")
    wid = c * NS + s
    ebase = wid * (BPW * CHUNK)

    pltpu.sync_copy(src_hbm.at[pl.ds(ebase, BPW * CHUNK)], srcbig)
    pltpu.sync_copy(ew_hbm.at[pl.ds(ebase, BPW * CHUNK)], ewbig)
    pltpu.sync_copy(zeros_hbm.at[pl.ds(s * ROWS_PER_SUB, ROWS_PER_SUB)],
                    acc_sh.at[pl.ds(s * ROWS_PER_SUB, ROWS_PER_SUB)])
    plsc.subcore_barrier()

    def stage(i):
        b = i % _ARING
        pltpu.async_copy(dst_hbm.at[pl.ds(ebase + i * CHUNK, CHUNK)],
                         dbufs[b], sds[b])

    def wait_stage(i):
        b = i % _ARING
        pltpu.make_async_copy(dst_hbm.at[pl.ds(ebase + i * CHUNK, CHUNK)],
                              dbufs[b], sds[b]).wait()

    def gather(i):
        b = i % _ARING
        pltpu.async_copy(g_hbm.at[srcbig.at[pl.ds(i * CHUNK, CHUNK)]],
                         rbufs[b], sgs[b])

    def wait_gather(i):
        b = i % _ARING
        pltpu.make_async_copy(g_hbm.at[srcbig.at[pl.ds(i * CHUNK, CHUNK)]],
                              rbufs[b], sgs[b]).wait()

    def scatter(i):
        b = i % _ARING
        pltpu.async_copy(rbufs[b], acc_sh.at[dbufs[b]], sss[b], add=True)

    def wait_scatter(i):
        b = i % _ARING
        pltpu.make_async_copy(rbufs[b], acc_sh.at[dbufs[b]], sss[b]).wait()

    for i in range(4):
        stage(i)
        gather(i)

    for i in range(BPW):
        wait_gather(i)
        rb = rbufs[i % _ARING]

        def scale16(k, _, i=i, rb=rb):
            v16 = ewbig[pl.ds(i * CHUNK + k * 16, 16)]
            for j in range(16):
                wv = jnp.take_along_axis(
                    v16, jnp.full((16,), j, jnp.int32), axis=0)
                e = k * 16 + j
                rb[e, :] = rb[e, :] * wv
            return 0

        lax.fori_loop(0, CHUNK // 16, scale16, 0)
        wait_stage(i)
        scatter(i)
        if i + 4 < BPW:
            if i - 4 >= 0:
                wait_scatter(i - 4)
            stage(i + 4)
            gather(i + 4)

    for i in range(BPW - _ARING, BPW):
        wait_scatter(i)
    plsc.subcore_barrier()
    pltpu.sync_copy(acc_sh.at[pl.ds(s * ROWS_PER_SUB, ROWS_PER_SUB)],
                    out_hbm.at[c, pl.ds(s * ROWS_PER_SUB, ROWS_PER_SUB)])


def _emb_body(cat_ref, wemb_ref, bemb_ref, out_ref):
    c2 = cat_ref[0]
    w = wemb_ref[0]
    out_ref[0] = lax.dot_general(
        c2, w, (((1,), (1,)), ((), ())),
        preferred_element_type=jnp.float32) + bemb_ref[0]


def _front_body(degpt_ref, numx_ref, emb_ref, w1_ref, g1_ref, dis_ref):
    d = degpt_ref[:, 0:1] + degpt_ref[:, 1:2] + 1.0
    dis = jnp.where(d > 0, lax.rsqrt(jnp.maximum(d, 1e-12)), 0.0)
    dis_ref[...] = dis
    h_main = lax.dot_general(
        numx_ref[...], w1_ref[...], (((1,), (1,)), ((), ())),
        preferred_element_type=jnp.float32, precision=HIGHEST)
    h_emb = lax.dot_general(
        emb_ref[...], w1_ref[...], (((1,), (1,)), ((), ())),
        preferred_element_type=jnp.float32, precision=HIGHEST)
    h = jnp.concatenate([h_main, h_emb], axis=0)
    g1_ref[...] = dis[:N] * h


def _layer2_body(pp_ref, g1_ref, dis_ref, b1_ref, w2_ref, out_ref):
    q = pp_ref[0] + pp_ref[1] + g1_ref[...]
    x1 = jnp.maximum(dis_ref[...] * q + b1_ref[...], 0.0)
    h2 = lax.dot_general(
        x1, w2_ref[...], (((1,), (1,)), ((), ())),
        preferred_element_type=jnp.float32, precision=HIGHEST)
    g2 = dis_ref[...] * h2
    out_ref[...] = jnp.concatenate(
        [g2, jnp.zeros_like(g2)], axis=1)


def _x2_body(pp_ref, g2_ref, dis_ref, b2_ref, out_ref):
    q = (pp_ref[0] + pp_ref[1] + g2_ref[...])[:, :F2]
    out_ref[...] = jnp.maximum(dis_ref[...] * q + b2_ref[...], 0.0)


def _fc_body(u_ref, wfc_ref, vo_ref, a_ref, bfc_ref, out_ref):
    s_row = lax.dot_general(
        u_ref[...], wfc_ref[...], (((1,), (1,)), ((), ())),
        preferred_element_type=jnp.float32)
    va = lax.dot_general(
        vo_ref[...], a_ref[...], (((1,), (1,)), ((), ())),
        preferred_element_type=jnp.float32, precision=HIGHEST)
    t = 1.1 * (va + s_row + bfc_ref[...])
    out_ref[...] = (jnp.maximum(t, 0.0) + jnp.log1p(jnp.exp(-jnp.abs(t)))) / 1.1


def kernel(num_x, cat_x, edge_index, edge_weights, vanilla_out, prob_dist,
           W1, b1, W2, b2, Wemb, bemb, Wfc, bfc):
    f32 = jnp.float32
    src = edge_index[0]
    dst = edge_index[1]
    pad = EPAD - E
    src_p = jnp.pad(src, (0, pad))
    dst_p = jnp.pad(dst, (0, pad))
    ew_p = jnp.pad(edge_weights, (0, pad))
    zeros1 = jnp.zeros((NPAD,), f32)
    zeros2 = jnp.zeros((NPAD, F1), f32)

    emb = pl.pallas_call(
        _emb_body,
        grid=(26,),
        in_specs=[
            pl.BlockSpec((1, 1, 1000), lambda f: (f, 0, 0)),
            pl.BlockSpec((1, 128, 1000), lambda f: (f, 0, 0)),
            pl.BlockSpec((1, 1, 128), lambda f: (f, 0, 0)),
        ],
        out_specs=pl.BlockSpec((1, 1, 128), lambda f: (f, 0, 0)),
        out_shape=jax.ShapeDtypeStruct((26, 1, 128), f32),
    )(cat_x.reshape(26, 1, 1000), Wemb, bemb.reshape(26, 1, 128))

    degp_t = jnp.transpose(_deg_kernel(dst_p, ew_p, zeros1))
    RB = 2000
    g1, dis_col_p = pl.pallas_call(
        _front_body,
        out_shape=(jax.ShapeDtypeStruct((N, F1), f32),
                   jax.ShapeDtypeStruct((NPAD, 1), f32)),
    )(degp_t, num_x, emb.reshape(26, 128), W1)
    dis_col = dis_col_p[:N]

    p1 = _agg_kernel(g1, src_p, dst_p, ew_p, zeros2)[:, :N]

    g2p = pl.pallas_call(
        _layer2_body,
        grid=(N // RB,),
        in_specs=[
            pl.BlockSpec((NC, RB, F1), lambda i: (0, i, 0)),
            pl.BlockSpec((RB, F1), lambda i: (i, 0)),
            pl.BlockSpec((RB, 1), lambda i: (i, 0)),
            pl.BlockSpec((1, F1), lambda i: (0, 0)),
            pl.BlockSpec((F2, F1), lambda i: (0, 0)),
        ],
        out_specs=pl.BlockSpec((RB, F1), lambda i: (i, 0)),
        out_shape=jax.ShapeDtypeStruct((N, F1), f32),
    )(p1, g1, dis_col, b1.reshape(1, F1), W2)

    p2 = _agg_kernel(g2p, src_p, dst_p, ew_p, zeros2)[:, :N]

    x2 = pl.pallas_call(
        _x2_body,
        grid=(N // RB,),
        in_specs=[
            pl.BlockSpec((NC, RB, F1), lambda i: (0, i, 0)),
            pl.BlockSpec((RB, F1), lambda i: (i, 0)),
            pl.BlockSpec((RB, 1), lambda i: (i, 0)),
            pl.BlockSpec((1, F2), lambda i: (0, 0)),
        ],
        out_specs=pl.BlockSpec((RB, F2), lambda i: (i, 0)),
        out_shape=jax.ShapeDtypeStruct((N, F2), f32),
    )(p2, g2p, dis_col, b2.reshape(1, F2))

    u = jnp.concatenate(
        [jnp.zeros((1, NCLS), f32), x2.reshape(1, N * F2), prob_dist], axis=1)
    A = Wfc[:, :NCLS]

    out = pl.pallas_call(
        _fc_body,
        out_shape=jax.ShapeDtypeStruct((vanilla_out.shape[0], NCLS), f32),
    )(u, Wfc, vanilla_out, A, bfc.reshape(1, NCLS))
    return out


# speedup vs baseline: 1.0096x; 1.0096x over previous
import functools

import jax
import jax.numpy as jnp
from jax import lax
from jax.experimental import pallas as pl
from jax.experimental.pallas import tpu as pltpu
from jax.experimental.pallas import tpu_sc as plsc

N = 10000
E = 160000
F1 = 16
F2 = 8
NCLS = 10
NC = 2
NS = 16
NW = NC * NS
CHUNK = 128
EPAD = 163840
BPW = EPAD // (NW * CHUNK)
BPS = EPAD // (NS * CHUNK)
NPAD = 10240
ROWS_PER_SUB = NPAD // NS

_mesh = plsc.VectorSubcoreMesh(core_axis_name="c", subcore_axis_name="s")

HIGHEST = jax.lax.Precision.HIGHEST


_RING = 8


@functools.partial(
    pl.kernel,
    out_type=jax.ShapeDtypeStruct((NPAD,), jnp.float32),
    mesh=_mesh,
    scratch_types=(
        [pltpu.VMEM((BPS * CHUNK,), jnp.float32),
         pltpu.VMEM((ROWS_PER_SUB,), jnp.float32)]
        + [pltpu.VMEM((CHUNK,), jnp.int32) for _ in range(_RING)]
        + [pltpu.SemaphoreType.DMA for _ in range(2 * _RING)]
        + [pltpu.VMEM_SHARED((NPAD,), jnp.float32)]
    ),
)
def _degdis_kernel(dst_hbm, ew_hbm, zeros_hbm, dis_hbm, ewbig, dbuf, *rest):
    dbufs = rest[:_RING]
    sds = rest[_RING:2 * _RING]
    sss = rest[2 * _RING:3 * _RING]
    acc_sh = rest[3 * _RING]
    c = lax.axis_index("c")
    s = lax.axis_index("s")
    ebase = s * (BPS * CHUNK)
    rbase = s * ROWS_PER_SUB

    pltpu.sync_copy(ew_hbm.at[pl.ds(ebase, BPS * CHUNK)], ewbig)
    pltpu.sync_copy(zeros_hbm.at[pl.ds(rbase, ROWS_PER_SUB)],
                    acc_sh.at[pl.ds(rbase, ROWS_PER_SUB)])
    plsc.subcore_barrier()

    def stage(i):
        b = i % _RING
        pltpu.async_copy(dst_hbm.at[pl.ds(ebase + i * CHUNK, CHUNK)],
                         dbufs[b], sds[b])

    def wait_stage(i):
        b = i % _RING
        pltpu.make_async_copy(dst_hbm.at[pl.ds(ebase + i * CHUNK, CHUNK)],
                              dbufs[b], sds[b]).wait()

    def scatter(i):
        b = i % _RING
        pltpu.async_copy(ewbig.at[pl.ds(i * CHUNK, CHUNK)],
                         acc_sh.at[dbufs[b]], sss[b], add=True)

    def wait_scatter(i):
        b = i % _RING
        pltpu.make_async_copy(ewbig.at[pl.ds(i * CHUNK, CHUNK)],
                              acc_sh.at[dbufs[b]], sss[b]).wait()

    for i in range(4):
        stage(i)
    for i in range(BPS):
        if i + 4 < BPS:
            if i - 4 >= 0:
                wait_scatter(i - 4)
            stage(i + 4)
        wait_stage(i)
        scatter(i)
    for i in range(BPS - _RING, BPS):
        wait_scatter(i)
    plsc.subcore_barrier()

    pltpu.sync_copy(acc_sh.at[pl.ds(rbase, ROWS_PER_SUB)], dbuf)

    def rsq16(k, _):
        d = dbuf[pl.ds(k * 16, 16)] + 1.0
        t = jnp.maximum(d, 1e-12)
        i0 = lax.bitcast_convert_type(t, jnp.int32)
        i0 = jnp.int32(0x5F3759DF) - lax.shift_right_arithmetic(
            i0, jnp.int32(1))
        y = lax.bitcast_convert_type(i0, jnp.float32)
        for _ in range(3):
            y = y * (1.5 - 0.5 * t * y * y)
        dbuf[pl.ds(k * 16, 16)] = jnp.where(d > 0, y, 0.0)
        return 0

    lax.fori_loop(0, ROWS_PER_SUB // 16, rsq16, 0)

    @pl.when(c == 0)
    def _():
        pltpu.sync_copy(dbuf, dis_hbm.at[pl.ds(rbase, ROWS_PER_SUB)])


def _make_agg_kernel(scale_by_dis):
    @functools.partial(
        pl.kernel,
        out_type=jax.ShapeDtypeStruct((NC, NPAD, F1), jnp.float32),
        mesh=_mesh,
        scratch_types=(
            [pltpu.VMEM((BPW * CHUNK,), jnp.int32),
             pltpu.VMEM((BPW * CHUNK,), jnp.float32)]
            + [pltpu.VMEM((CHUNK,), jnp.int32) for _ in range(_RING)]
            + [pltpu.VMEM((CHUNK, F1), jnp.float32) for _ in range(_RING)]
            + [pltpu.VMEM((CHUNK,), jnp.float32) for _ in range(_RING)]
            + [pltpu.SemaphoreType.DMA for _ in range(4 * _RING)]
            + [pltpu.VMEM_SHARED((NPAD, F1), jnp.float32)]
        ),
        compiler_params=pltpu.CompilerParams(use_tc_tiling_on_sc=False),
    )
    def agg(g_hbm, dis_hbm, src_hbm, dst_hbm, ew_hbm, zeros_hbm, out_hbm,
            srcbig, ewbig, *rest):
        dbufs = rest[:_RING]
        rbufs = rest[_RING:2 * _RING]
        vbufs = rest[2 * _RING:3 * _RING]
        sds = rest[3 * _RING:4 * _RING]
        sgs = rest[4 * _RING:5 * _RING]
        sss = rest[5 * _RING:6 * _RING]
        svs = rest[6 * _RING:7 * _RING]
        acc_sh = rest[7 * _RING]
        c = lax.axis_index("c")
        s = lax.axis_index("s")
        wid = c * NS + s
        ebase = wid * (BPW * CHUNK)

        pltpu.sync_copy(src_hbm.at[pl.ds(ebase, BPW * CHUNK)], srcbig)
        pltpu.sync_copy(ew_hbm.at[pl.ds(ebase, BPW * CHUNK)], ewbig)
        pltpu.sync_copy(zeros_hbm.at[pl.ds(s * ROWS_PER_SUB, ROWS_PER_SUB)],
                        acc_sh.at[pl.ds(s * ROWS_PER_SUB, ROWS_PER_SUB)])
        plsc.subcore_barrier()

        def stage(i):
            b = i % _RING
            pltpu.async_copy(dst_hbm.at[pl.ds(ebase + i * CHUNK, CHUNK)],
                             dbufs[b], sds[b])

        def wait_stage(i):
            b = i % _RING
            pltpu.make_async_copy(dst_hbm.at[pl.ds(ebase + i * CHUNK, CHUNK)],
                                  dbufs[b], sds[b]).wait()

        def gather(i):
            b = i % _RING
            pltpu.async_copy(g_hbm.at[srcbig.at[pl.ds(i * CHUNK, CHUNK)]],
                             rbufs[b], sgs[b])
            if scale_by_dis:
                pltpu.async_copy(
                    dis_hbm.at[srcbig.at[pl.ds(i * CHUNK, CHUNK)]],
                    vbufs[b], svs[b])

        def wait_gather(i):
            b = i % _RING
            pltpu.make_async_copy(g_hbm.at[srcbig.at[pl.ds(i * CHUNK, CHUNK)]],
                                  rbufs[b], sgs[b]).wait()
            if scale_by_dis:
                pltpu.make_async_copy(
                    dis_hbm.at[srcbig.at[pl.ds(i * CHUNK, CHUNK)]],
                    vbufs[b], svs[b]).wait()

        def scatter(i):
            b = i % _RING
            pltpu.async_copy(rbufs[b], acc_sh.at[dbufs[b]], sss[b], add=True)

        def wait_scatter(i):
            b = i % _RING
            pltpu.make_async_copy(rbufs[b], acc_sh.at[dbufs[b]],
                                  sss[b]).wait()

        for i in range(4):
            stage(i)
            gather(i)

        for i in range(BPW):
            wait_gather(i)
            b = i % _RING
            rb = rbufs[b]
            vb = vbufs[b]

            def scale16(k, _, i=i, rb=rb, vb=vb):
                v16 = ewbig[pl.ds(i * CHUNK + k * 16, 16)]
                if scale_by_dis:
                    v16 = v16 * vb[pl.ds(k * 16, 16)]
                for j in range(16):
                    wv = jnp.take_along_axis(
                        v16, jnp.full((16,), j, jnp.int32), axis=0)
                    e = k * 16 + j
                    rb[e, :] = rb[e, :] * wv
                return 0

            lax.fori_loop(0, CHUNK // 16, scale16, 0)
            wait_stage(i)
            scatter(i)
            if i + 4 < BPW:
                if i - 4 >= 0:
                    wait_scatter(i - 4)
                stage(i + 4)
                gather(i + 4)

        for i in range(BPW - _RING, BPW):
            wait_scatter(i)
        plsc.subcore_barrier()
        pltpu.sync_copy(acc_sh.at[pl.ds(s * ROWS_PER_SUB, ROWS_PER_SUB)],
                        out_hbm.at[c, pl.ds(s * ROWS_PER_SUB, ROWS_PER_SUB)])

    return agg


_agg2_kernel = _make_agg_kernel(scale_by_dis=False)


def _pre_body(cat_ref, wemb_ref, bemb_ref, numx_ref, w1_ref, h1_ref):
    emb = lax.dot_general(
        cat_ref[...], wemb_ref[...], (((2,), (2,)), ((0,), (0,))),
        preferred_element_type=jnp.float32)
    emb = emb[:, 0, :] + bemb_ref[...]
    h_main = lax.dot_general(
        numx_ref[...], w1_ref[...], (((1,), (1,)), ((), ())),
        preferred_element_type=jnp.float32, precision=HIGHEST)
    h_emb = lax.dot_general(
        emb, w1_ref[...], (((1,), (1,)), ((), ())),
        preferred_element_type=jnp.float32, precision=HIGHEST)
    h1_ref[...] = jnp.concatenate([h_main, h_emb], axis=0)


def _g1_body(h1_ref, dis_ref, out_ref):
    out_ref[...] = dis_ref[...] * h1_ref[...]


def _mid_body(pp_ref, g1_ref, dis_ref, b1_ref, w2_ref, out_ref):
    dis = dis_ref[...]
    q = pp_ref[0] + pp_ref[1] + g1_ref[...]
    x1 = jnp.maximum(dis * q + b1_ref[...], 0.0)
    h2 = lax.dot_general(
        x1, w2_ref[...], (((1,), (1,)), ((), ())),
        preferred_element_type=jnp.float32, precision=HIGHEST)
    g2 = dis * h2
    out_ref[...] = jnp.concatenate([g2, jnp.zeros_like(g2)], axis=1)


def _x2_body(pp_ref, g2_ref, dis_ref, b2_ref, out_ref):
    q = (pp_ref[0] + pp_ref[1] + g2_ref[...])[:, :F2]
    out_ref[...] = jnp.maximum(dis_ref[...] * q + b2_ref[...], 0.0)


def _fc_body(u_ref, wfc_ref, vo_ref, a_ref, bfc_ref, out_ref):
    s_row = lax.dot_general(
        u_ref[...], wfc_ref[...], (((1,), (1,)), ((), ())),
        preferred_element_type=jnp.float32)
    va = lax.dot_general(
        vo_ref[...], a_ref[...], (((1,), (1,)), ((), ())),
        preferred_element_type=jnp.float32, precision=HIGHEST)
    t = 1.1 * (va + s_row + bfc_ref[...])
    out_ref[...] = (jnp.maximum(t, 0.0) + jnp.log1p(jnp.exp(-jnp.abs(t)))) / 1.1


def kernel(num_x, cat_x, edge_index, edge_weights, vanilla_out, prob_dist,
           W1, b1, W2, b2, Wemb, bemb, Wfc, bfc):
    f32 = jnp.float32
    pad = EPAD - E
    src_p = jnp.pad(edge_index[0], (0, pad))
    dst_p = jnp.pad(edge_index[1], (0, pad))
    ew_p = jnp.pad(edge_weights, (0, pad))
    zeros1 = jnp.zeros((NPAD,), f32)
    zeros2 = jnp.zeros((NPAD, F1), f32)

    dis = _degdis_kernel(dst_p, ew_p, zeros1)
    dis_col = dis.reshape(NPAD, 1)[:N]

    h1 = pl.pallas_call(
        _pre_body,
        out_shape=jax.ShapeDtypeStruct((N, F1), f32),
    )(cat_x.reshape(26, 1, 1000), Wemb, bemb, num_x, W1)

    RBG = 2000
    g1 = pl.pallas_call(
        _g1_body,
        grid=(N // RBG,),
        in_specs=[
            pl.BlockSpec((RBG, F1), lambda i: (i, 0)),
            pl.BlockSpec((RBG, 1), lambda i: (i, 0)),
        ],
        out_specs=pl.BlockSpec((RBG, F1), lambda i: (i, 0)),
        out_shape=jax.ShapeDtypeStruct((N, F1), f32),
    )(h1, dis_col)
    p1 = _agg2_kernel(g1, dis, src_p, dst_p, ew_p, zeros2)

    RB = 2000
    g2p = pl.pallas_call(
        _mid_body,
        grid=(N // RB,),
        in_specs=[
            pl.BlockSpec((NC, RB, F1), lambda i: (0, i, 0)),
            pl.BlockSpec((RB, F1), lambda i: (i, 0)),
            pl.BlockSpec((RB, 1), lambda i: (i, 0)),
            pl.BlockSpec((1, F1), lambda i: (0, 0)),
            pl.BlockSpec((F2, F1), lambda i: (0, 0)),
        ],
        out_specs=pl.BlockSpec((RB, F1), lambda i: (i, 0)),
        out_shape=jax.ShapeDtypeStruct((N, F1), f32),
    )(p1[:, :N], g1, dis_col, b1.reshape(1, F1), W2)

    p2 = _agg2_kernel(g2p, dis, src_p, dst_p, ew_p, zeros2)

    x2 = pl.pallas_call(
        _x2_body,
        grid=(N // RB,),
        in_specs=[
            pl.BlockSpec((NC, RB, F1), lambda i: (0, i, 0)),
            pl.BlockSpec((RB, F1), lambda i: (i, 0)),
            pl.BlockSpec((RB, 1), lambda i: (i, 0)),
            pl.BlockSpec((1, F2), lambda i: (0, 0)),
        ],
        out_specs=pl.BlockSpec((RB, F2), lambda i: (i, 0)),
        out_shape=jax.ShapeDtypeStruct((N, F2), f32),
    )(p2[:, :N], g2p, dis_col, b2.reshape(1, F2))

    u = jnp.concatenate(
        [jnp.zeros((1, NCLS), f32), x2.reshape(1, N * F2), prob_dist], axis=1)
    A = Wfc[:, :NCLS]
    out = pl.pallas_call(
        _fc_body,
        out_shape=jax.ShapeDtypeStruct((vanilla_out.shape[0], NCLS), f32),
    )(u, Wfc, vanilla_out, A, bfc.reshape(1, NCLS))
    return out
